# Initial kernel scaffold; baseline (speedup 1.0000x reference)
import functools

import jax
import jax.numpy as jnp
from jax import lax
from jax.experimental import pallas as pl
from jax.experimental.pallas import tpu as pltpu
from jax.experimental.pallas import tpu_sc as plsc

N = 50000
E = 800000
D = 64

NC = 2
NS = 16
L = 16

BLK = 512
GRID = 98
HROWS = GRID * BLK
NPAD = 51200
ROWS_PT = NPAD // NS
TRASH = NPAD - 1
SRC_PAD = HROWS - 1
CH = 128
GRP = 20
NGRP = 20
EPT = CH * GRP * NGRP
EPAD = EPT * NS
ZROWS = 320

_mesh = plsc.VectorSubcoreMesh(core_axis_name="c", subcore_axis_name="s")


@functools.partial(
    pl.kernel,
    out_type=jax.ShapeDtypeStruct((NC, NPAD, 16), jnp.float32),
    mesh=_mesh,
    scratch_types=[
        pltpu.VMEM_SHARED((NPAD, 16), jnp.float32),
        pltpu.VMEM((GRP, CH), jnp.int32),
        pltpu.VMEM((CH, 16), jnp.float32),
        pltpu.VMEM((ZROWS, 16), jnp.float32),
    ],
)
def _hist_kernel(edges, degs, hist, sbuf, ones, zbuf):
    c = lax.axis_index("c")
    s = lax.axis_index("s")
    one = jnp.ones((L,), jnp.float32)
    zero = jnp.zeros((L,), jnp.float32)

    @pl.loop(0, CH)
    def _(i):
        ones[i, :] = one

    @pl.loop(0, ZROWS)
    def _(i):
        zbuf[i, :] = zero

    @pl.loop(0, ROWS_PT // ZROWS)
    def _(i):
        pltpu.sync_copy(zbuf, hist.at[pl.ds(s * ROWS_PT + i * ZROWS, ZROWS)])

    plsc.subcore_barrier()

    @pl.loop(0, NGRP)
    def _(g):
        pltpu.sync_copy(edges.at[c, s * NGRP + g], sbuf)
        for j in range(GRP):
            pltpu.sync_copy(ones, hist.at[sbuf.at[j]], add=True)

    plsc.subcore_barrier()

    @pl.loop(0, ROWS_PT // ZROWS)
    def _(i):
        r0 = s * ROWS_PT + i * ZROWS
        pltpu.sync_copy(hist.at[pl.ds(r0, ZROWS)], degs.at[c, pl.ds(r0, ZROWS)])


@functools.partial(
    pl.kernel,
    out_type=jax.ShapeDtypeStruct((NC, NPAD, 32), jnp.float32),
    mesh=_mesh,
    scratch_types=[
        pltpu.VMEM_SHARED((NPAD, 32), jnp.float32),
        pltpu.VMEM((GRP, CH), jnp.int32),
        pltpu.VMEM((GRP, CH), jnp.int32),
        pltpu.VMEM((2, CH), jnp.int32),
        pltpu.VMEM((2, CH, 32), jnp.float32),
        pltpu.VMEM((ZROWS, 32), jnp.float32),
        pltpu.SemaphoreType.DMA((2,)),
    ],
)
def _gs_kernel(hi, edges, agg_out, agg, sbuf, dbuf, tbuf, gbuf, zbuf, gsem):
    c = lax.axis_index("c")
    s = lax.axis_index("s")
    zero = jnp.zeros((L,), jnp.float32)

    @pl.loop(0, ZROWS)
    def _(i):
        zbuf[i, pl.ds(0, L)] = zero
        zbuf[i, pl.ds(L, L)] = zero

    @pl.loop(0, ROWS_PT // ZROWS)
    def _(i):
        pltpu.sync_copy(zbuf, agg.at[pl.ds(s * ROWS_PT + i * ZROWS, ZROWS)])

    plsc.subcore_barrier()

    @pl.loop(0, NGRP)
    def _(g):
        pltpu.sync_copy(edges.at[0, s * NGRP + g], sbuf)
        pltpu.sync_copy(edges.at[1, s * NGRP + g], dbuf)

        descs = {}

        def start(j, bb):
            for k in range(CH // L):
                v = sbuf[j, pl.ds(k * L, L)]
                tbuf[bb, pl.ds(k * L, L)] = v * 2 + c
            descs[bb] = pltpu.async_copy(hi.at[tbuf.at[bb]], gbuf.at[bb],
                                         gsem.at[bb])

        start(0, 0)
        for j in range(GRP):
            bb = j % 2
            if j + 1 < GRP:
                start(j + 1, (j + 1) % 2)
            descs[bb].wait()
            pltpu.sync_copy(gbuf.at[bb], agg.at[dbuf.at[j]], add=True)

    plsc.subcore_barrier()

    @pl.loop(0, ROWS_PT // ZROWS)
    def _(i):
        r0 = s * ROWS_PT + i * ZROWS
        pltpu.sync_copy(agg.at[pl.ds(r0, ZROWS)],
                        agg_out.at[c, pl.ds(r0, ZROWS)])


def _mm_body(x_ref, w_ref, d_ref, o_ref):
    x = x_ref[...]
    w = w_ref[...]
    deg = d_ref[0, :, 0:1]
    norm = jnp.where(deg > 0, lax.rsqrt(jnp.maximum(deg, 1e-12)), 0.0)
    y = jnp.dot(x, w, preferred_element_type=jnp.float32) * norm
    o_ref[...] = y.reshape(BLK, 2, 32)


def _out_body(a_ref, d_ref, b_ref, o_ref):
    agg = jnp.concatenate([a_ref[0], a_ref[1]], axis=-1)
    deg = d_ref[0, :, 0:1]
    norm = jnp.where(deg > 0, lax.rsqrt(jnp.maximum(deg, 1e-12)), 0.0)
    o_ref[...] = agg * norm + b_ref[...]


def kernel(node_embeddings, W, b, edge_index):
    pad_src = jnp.full((EPAD - E,), SRC_PAD, jnp.int32)
    pad_dst = jnp.full((EPAD - E,), TRASH, jnp.int32)
    srcp = jnp.concatenate([edge_index[0], pad_src]).reshape(NS * NGRP, GRP, CH)
    dstp = jnp.concatenate([edge_index[1], pad_dst]).reshape(NS * NGRP, GRP, CH)
    edges2 = jnp.stack([srcp, dstp])

    degs = _hist_kernel(edges2)

    hi = pl.pallas_call(
        _mm_body,
        grid=(GRID,),
        in_specs=[
            pl.BlockSpec((BLK, D), lambda i: (i, 0)),
            pl.BlockSpec((D, D), lambda i: (0, 0)),
            pl.BlockSpec((1, BLK, 16), lambda i: (0, i, 0)),
        ],
        out_specs=pl.BlockSpec((BLK, 2, 32), lambda i: (i, 0, 0)),
        out_shape=jax.ShapeDtypeStruct((HROWS, 2, 32), jnp.float32),
    )(node_embeddings, W, degs)

    aggi = _gs_kernel(hi.reshape(HROWS * 2, 32), edges2)

    out = pl.pallas_call(
        _out_body,
        grid=(GRID,),
        in_specs=[
            pl.BlockSpec((2, BLK, 32), lambda i: (0, i, 0)),
            pl.BlockSpec((1, BLK, 16), lambda i: (1, i, 0)),
            pl.BlockSpec((1, D), lambda i: (0, 0)),
        ],
        out_specs=pl.BlockSpec((BLK, D), lambda i: (i, 0)),
        out_shape=jax.ShapeDtypeStruct((N, D), jnp.float32),
    )(aggi, degs, b.reshape(1, D))
    return out


# speedup vs baseline: 5.1973x; 5.1973x over previous
import functools

import jax
import jax.numpy as jnp
from jax import lax
from jax.experimental import pallas as pl
from jax.experimental.pallas import tpu as pltpu
from jax.experimental.pallas import tpu_sc as plsc

N = 50000
E = 800000
D = 64

NC = 2
NS = 16
L = 16

BLK = 512
GRID = 98
HROWS = GRID * BLK
NPAD = 51200
ROWS_PT = NPAD // NS
TRASH = NPAD - 1
SRC_PAD = HROWS - 1
CH = 128
GRP = 20
NGRP = 20
EPT = CH * GRP * NGRP
EPAD = EPT * NS
ZROWS = 320

_mesh = plsc.VectorSubcoreMesh(core_axis_name="c", subcore_axis_name="s")


@functools.partial(
    pl.kernel,
    out_type=jax.ShapeDtypeStruct((NC, NPAD, 16), jnp.float32),
    mesh=_mesh,
    scratch_types=[
        pltpu.VMEM_SHARED((NPAD, 16), jnp.float32),
        pltpu.VMEM((GRP, CH), jnp.int32),
        pltpu.VMEM((CH, 16), jnp.float32),
        pltpu.VMEM((ZROWS, 16), jnp.float32),
    ],
    compiler_params=pltpu.CompilerParams(use_tc_tiling_on_sc=False),
)
def _hist_kernel(edges, degs, hist, sbuf, ones, zbuf):
    c = lax.axis_index("c")
    s = lax.axis_index("s")
    one = jnp.ones((L,), jnp.float32)
    zero = jnp.zeros((L,), jnp.float32)

    @pl.loop(0, CH)
    def _(i):
        ones[i, :] = one

    @pl.loop(0, ZROWS)
    def _(i):
        zbuf[i, :] = zero

    @pl.loop(0, ROWS_PT // ZROWS)
    def _(i):
        pltpu.sync_copy(zbuf, hist.at[pl.ds(s * ROWS_PT + i * ZROWS, ZROWS)])

    plsc.subcore_barrier()

    @pl.loop(0, NGRP)
    def _(g):
        pltpu.sync_copy(edges.at[c, s * NGRP + g], sbuf)
        for j in range(GRP):
            pltpu.sync_copy(ones, hist.at[sbuf.at[j]], add=True)

    plsc.subcore_barrier()

    @pl.loop(0, ROWS_PT // ZROWS)
    def _(i):
        r0 = s * ROWS_PT + i * ZROWS
        pltpu.sync_copy(hist.at[pl.ds(r0, ZROWS)], degs.at[c, pl.ds(r0, ZROWS)])


@functools.partial(
    pl.kernel,
    out_type=jax.ShapeDtypeStruct((NC, NPAD, 32), jnp.float32),
    mesh=_mesh,
    scratch_types=[
        pltpu.VMEM_SHARED((NPAD, 32), jnp.float32),
        pltpu.VMEM((GRP, CH), jnp.int32),
        pltpu.VMEM((GRP, CH), jnp.int32),
        pltpu.VMEM((2, CH), jnp.int32),
        pltpu.VMEM((2, CH, 32), jnp.float32),
        pltpu.VMEM((ZROWS, 32), jnp.float32),
        pltpu.SemaphoreType.DMA((2,)),
    ],
    compiler_params=pltpu.CompilerParams(use_tc_tiling_on_sc=False),
)
def _gs_kernel(hi, edges, agg_out, agg, sbuf, dbuf, tbuf, gbuf, zbuf, gsem):
    c = lax.axis_index("c")
    s = lax.axis_index("s")
    zero = jnp.zeros((L,), jnp.float32)

    @pl.loop(0, ZROWS)
    def _(i):
        zbuf[i, pl.ds(0, L)] = zero
        zbuf[i, pl.ds(L, L)] = zero

    @pl.loop(0, ROWS_PT // ZROWS)
    def _(i):
        pltpu.sync_copy(zbuf, agg.at[pl.ds(s * ROWS_PT + i * ZROWS, ZROWS)])

    plsc.subcore_barrier()

    @pl.loop(0, NGRP)
    def _(g):
        pltpu.sync_copy(edges.at[0, s * NGRP + g], sbuf)
        pltpu.sync_copy(edges.at[1, s * NGRP + g], dbuf)

        descs = {}

        def start(j, bb):
            for k in range(CH // L):
                v = sbuf[j, pl.ds(k * L, L)]
                tbuf[bb, pl.ds(k * L, L)] = v * 2 + c
            descs[bb] = pltpu.async_copy(hi.at[tbuf.at[bb]], gbuf.at[bb],
                                         gsem.at[bb])

        start(0, 0)
        for j in range(GRP):
            bb = j % 2
            if j + 1 < GRP:
                start(j + 1, (j + 1) % 2)
            descs[bb].wait()
            pltpu.sync_copy(gbuf.at[bb], agg.at[dbuf.at[j]], add=True)

    plsc.subcore_barrier()

    @pl.loop(0, ROWS_PT // ZROWS)
    def _(i):
        r0 = s * ROWS_PT + i * ZROWS
        pltpu.sync_copy(agg.at[pl.ds(r0, ZROWS)],
                        agg_out.at[c, pl.ds(r0, ZROWS)])


def _mm_body(x_ref, w_ref, d_ref, o_ref):
    x = x_ref[...]
    w = w_ref[...]
    deg = d_ref[0, :, 0:1]
    norm = jnp.where(deg > 0, lax.rsqrt(jnp.maximum(deg, 1e-12)), 0.0)
    y = jnp.dot(x, w, preferred_element_type=jnp.float32) * norm
    o_ref[...] = y.reshape(BLK, 2, 32)


def _out_body(a_ref, d_ref, b_ref, o_ref):
    agg = jnp.concatenate([a_ref[0], a_ref[1]], axis=-1)
    deg = d_ref[0, :, 0:1]
    norm = jnp.where(deg > 0, lax.rsqrt(jnp.maximum(deg, 1e-12)), 0.0)
    o_ref[...] = agg * norm + b_ref[...]


def kernel(node_embeddings, W, b, edge_index):
    pad_src = jnp.full((EPAD - E,), SRC_PAD, jnp.int32)
    pad_dst = jnp.full((EPAD - E,), TRASH, jnp.int32)
    srcp = jnp.concatenate([edge_index[0], pad_src]).reshape(NS * NGRP, GRP, CH)
    dstp = jnp.concatenate([edge_index[1], pad_dst]).reshape(NS * NGRP, GRP, CH)
    edges2 = jnp.stack([srcp, dstp])

    degs = _hist_kernel(edges2)

    hi = pl.pallas_call(
        _mm_body,
        grid=(GRID,),
        in_specs=[
            pl.BlockSpec((BLK, D), lambda i: (i, 0)),
            pl.BlockSpec((D, D), lambda i: (0, 0)),
            pl.BlockSpec((1, BLK, 16), lambda i: (0, i, 0)),
        ],
        out_specs=pl.BlockSpec((BLK, 2, 32), lambda i: (i, 0, 0)),
        out_shape=jax.ShapeDtypeStruct((HROWS, 2, 32), jnp.float32),
    )(node_embeddings, W, degs)

    aggi = _gs_kernel(hi.reshape(HROWS * 2, 32), edges2)

    out = pl.pallas_call(
        _out_body,
        grid=(GRID,),
        in_specs=[
            pl.BlockSpec((2, BLK, 32), lambda i: (0, i, 0)),
            pl.BlockSpec((1, BLK, 16), lambda i: (1, i, 0)),
            pl.BlockSpec((1, D), lambda i: (0, 0)),
        ],
        out_specs=pl.BlockSpec((BLK, D), lambda i: (i, 0)),
        out_shape=jax.ShapeDtypeStruct((N, D), jnp.float32),
    )(aggi, degs, b.reshape(1, D))
    return out


# speedup vs baseline: 5.9005x; 1.1353x over previous
import functools

import jax
import jax.numpy as jnp
from jax import lax
from jax.experimental import pallas as pl
from jax.experimental.pallas import tpu as pltpu
from jax.experimental.pallas import tpu_sc as plsc

N = 50000
E = 800000
D = 64

NC = 2
NS = 16
L = 16

BLK = 6272
GRID = 8
HROWS = GRID * BLK
NPAD = 51200
ROWS_PT = NPAD // NS
TRASH = NPAD - 1
SRC_PAD = HROWS - 1
CH = 128
GRP = 10
NGRP = 40
EPT = CH * GRP * NGRP
EPAD = EPT * NS
ZROWS = 320
NBUF = 6
LEAD = 3
SBUF = 3

_mesh = plsc.VectorSubcoreMesh(core_axis_name="c", subcore_axis_name="s")
_sc_params = pltpu.CompilerParams(use_tc_tiling_on_sc=False)


@functools.partial(
    pl.kernel,
    out_type=jax.ShapeDtypeStruct((NC, NPAD, 16), jnp.float32),
    mesh=_mesh,
    scratch_types=[
        pltpu.VMEM_SHARED((NPAD, 16), jnp.float32),
        pltpu.VMEM((GRP, CH), jnp.int32),
        pltpu.VMEM((CH, 16), jnp.float32),
        pltpu.VMEM((ZROWS, 16), jnp.float32),
        pltpu.SemaphoreType.DMA,
    ],
    compiler_params=_sc_params,
)
def _hist_kernel(edges, degs, hist, sbuf, ones, zbuf, ssem):
    c = lax.axis_index("c")
    s = lax.axis_index("s")
    one = jnp.ones((L,), jnp.float32)
    zero = jnp.zeros((L,), jnp.float32)

    @pl.loop(0, CH)
    def _(i):
        ones[i, :] = one

    @pl.loop(0, ZROWS)
    def _(i):
        zbuf[i, :] = zero

    @pl.loop(0, ROWS_PT // ZROWS)
    def _(i):
        pltpu.sync_copy(zbuf, hist.at[pl.ds(s * ROWS_PT + i * ZROWS, ZROWS)])

    plsc.subcore_barrier()

    @pl.loop(0, NGRP)
    def _(g):
        pltpu.sync_copy(edges.at[c, s * NGRP + g], sbuf)
        descs = [
            pltpu.async_copy(ones, hist.at[sbuf.at[j]], ssem, add=True)
            for j in range(GRP)
        ]
        for d in descs:
            d.wait()

    plsc.subcore_barrier()

    @pl.loop(0, ROWS_PT // ZROWS)
    def _(i):
        r0 = s * ROWS_PT + i * ZROWS
        pltpu.sync_copy(hist.at[pl.ds(r0, ZROWS)], degs.at[c, pl.ds(r0, ZROWS)])


@functools.partial(
    pl.kernel,
    out_type=jax.ShapeDtypeStruct((NC, NPAD, 32), jnp.float32),
    mesh=_mesh,
    scratch_types=[
        pltpu.VMEM_SHARED((NPAD, 32), jnp.float32),
        pltpu.VMEM((GRP, CH), jnp.int32),
        pltpu.VMEM((GRP, CH), jnp.int32),
        pltpu.VMEM((NBUF, CH), jnp.int32),
        pltpu.VMEM((NBUF, CH, 32), jnp.float32),
        pltpu.SemaphoreType.DMA((NBUF,)),
        pltpu.SemaphoreType.DMA((SBUF,)),
    ],
    compiler_params=_sc_params,
)
def _gs_kernel(hi, edges, agg_out, agg, sbuf, dbuf, tbuf, gbuf,
               gsem, ssem):
    c = lax.axis_index("c")
    s = lax.axis_index("s")
    zero = jnp.zeros((L,), jnp.float32)

    @pl.loop(0, CH)
    def _(i):
        gbuf[0, i, pl.ds(0, L)] = zero
        gbuf[0, i, pl.ds(L, L)] = zero

    @pl.loop(0, ROWS_PT // CH)
    def _(i):
        pltpu.sync_copy(gbuf.at[0],
                        agg.at[pl.ds(s * ROWS_PT + i * CH, CH)])

    plsc.subcore_barrier()

    @pl.loop(0, NGRP)
    def _(g):
        pltpu.sync_copy(edges.at[0, s * NGRP + g], sbuf)
        pltpu.sync_copy(edges.at[1, s * NGRP + g], dbuf)

        gdesc = {}
        sdesc = {}

        def start_gather(q):
            bq = q % NBUF
            for k in range(CH // L):
                v = sbuf[q, pl.ds(k * L, L)]
                tbuf[bq, pl.ds(k * L, L)] = v * 2 + c
            gdesc[q] = pltpu.async_copy(hi.at[tbuf.at[bq]], gbuf.at[bq],
                                        gsem.at[bq])

        for q in range(LEAD):
            start_gather(q)
        for j in range(GRP):
            if j - SBUF >= 0:
                sdesc[j - SBUF].wait()
            q = j + LEAD
            if q < GRP:
                start_gather(q)
            gdesc[j].wait()
            sdesc[j] = pltpu.async_copy(gbuf.at[j % NBUF],
                                        agg.at[dbuf.at[j]],
                                        ssem.at[j % SBUF], add=True)
        for j in range(GRP - SBUF, GRP):
            sdesc[j].wait()

    plsc.subcore_barrier()

    @pl.loop(0, ROWS_PT // ZROWS)
    def _(i):
        r0 = s * ROWS_PT + i * ZROWS
        pltpu.sync_copy(agg.at[pl.ds(r0, ZROWS)],
                        agg_out.at[c, pl.ds(r0, ZROWS)])


def _mm_body(x_ref, w_ref, o_ref):
    o_ref[...] = jnp.dot(x_ref[...], w_ref[...],
                         preferred_element_type=jnp.float32)


def _scale_body(y_ref, d_ref, o_ref):
    deg = d_ref[0, :, 0:1]
    norm = jnp.where(deg > 0, lax.rsqrt(jnp.maximum(deg, 1e-12)), 0.0)
    o_ref[...] = (y_ref[...] * norm).reshape(BLK, 2, 32)


def _out_body(a_ref, d_ref, b_ref, o_ref):
    agg = jnp.concatenate([a_ref[0], a_ref[1]], axis=-1)
    deg = d_ref[0, :, 0:1]
    norm = jnp.where(deg > 0, lax.rsqrt(jnp.maximum(deg, 1e-12)), 0.0)
    o_ref[...] = agg * norm + b_ref[...]


def kernel(node_embeddings, W, b, edge_index):
    pad_src = jnp.full((EPAD - E,), SRC_PAD, jnp.int32)
    pad_dst = jnp.full((EPAD - E,), TRASH, jnp.int32)
    srcp = jnp.concatenate([edge_index[0], pad_src]).reshape(NS * NGRP, GRP, CH)
    dstp = jnp.concatenate([edge_index[1], pad_dst]).reshape(NS * NGRP, GRP, CH)
    edges2 = jnp.stack([srcp, dstp])

    degs = _hist_kernel(edges2)

    y = pl.pallas_call(
        _mm_body,
        grid=(GRID,),
        in_specs=[
            pl.BlockSpec((BLK, D), lambda i: (i, 0)),
            pl.BlockSpec((D, D), lambda i: (0, 0)),
        ],
        out_specs=pl.BlockSpec((BLK, D), lambda i: (i, 0)),
        out_shape=jax.ShapeDtypeStruct((HROWS, D), jnp.float32),
    )(node_embeddings, W)

    hi = pl.pallas_call(
        _scale_body,
        grid=(GRID,),
        in_specs=[
            pl.BlockSpec((BLK, D), lambda i: (i, 0)),
            pl.BlockSpec((1, BLK, 16), lambda i: (0, i, 0)),
        ],
        out_specs=pl.BlockSpec((BLK, 2, 32), lambda i: (i, 0, 0)),
        out_shape=jax.ShapeDtypeStruct((HROWS, 2, 32), jnp.float32),
    )(y, degs)

    aggi = _gs_kernel(hi.reshape(HROWS * 2, 32), edges2)

    out = pl.pallas_call(
        _out_body,
        grid=(GRID,),
        in_specs=[
            pl.BlockSpec((2, BLK, 32), lambda i: (0, i, 0)),
            pl.BlockSpec((1, BLK, 16), lambda i: (1, i, 0)),
            pl.BlockSpec((1, D), lambda i: (0, 0)),
        ],
        out_specs=pl.BlockSpec((BLK, D), lambda i: (i, 0)),
        out_shape=jax.ShapeDtypeStruct((N, D), jnp.float32),
    )(aggi, degs, b.reshape(1, D))
    return out


# speedup vs baseline: 7.0750x; 1.1990x over previous
import functools

import jax
import jax.numpy as jnp
from jax import lax
from jax.experimental import pallas as pl
from jax.experimental.pallas import tpu as pltpu
from jax.experimental.pallas import tpu_sc as plsc

N = 50000
E = 800000
D = 64

NC = 2
NS = 16
L = 16

BLK = 6272
GRID = 8
HROWS = GRID * BLK
NPAD = 51200
ROWS_PT = NPAD // NS
TRASH = NPAD - 1
SRC_PAD = HROWS - 1
CH = 128
GRP = 10
NGRP = 40
EPT = CH * GRP * NGRP
EPAD = EPT * NS
ZROWS = 320
NBUF = 6
LEAD = 3
SBUF = 3

_mesh = plsc.VectorSubcoreMesh(core_axis_name="c", subcore_axis_name="s")
_sc_params = pltpu.CompilerParams(use_tc_tiling_on_sc=False)


@functools.partial(
    pl.kernel,
    out_type=jax.ShapeDtypeStruct((NC, NPAD, 16), jnp.float32),
    mesh=_mesh,
    scratch_types=[
        pltpu.VMEM_SHARED((NPAD, 16), jnp.float32),
        pltpu.VMEM((GRP, CH), jnp.int32),
        pltpu.VMEM((CH, 16), jnp.float32),
        pltpu.VMEM((ZROWS, 16), jnp.float32),
        pltpu.SemaphoreType.DMA,
    ],
    compiler_params=_sc_params,
)
def _hist_kernel(edges, degs, hist, sbuf, ones, zbuf, ssem):
    c = lax.axis_index("c")
    s = lax.axis_index("s")
    one = jnp.ones((L,), jnp.float32)
    zero = jnp.zeros((L,), jnp.float32)

    @pl.loop(0, CH)
    def _(i):
        ones[i, :] = one

    @pl.loop(0, ZROWS)
    def _(i):
        zbuf[i, :] = zero

    @pl.loop(0, ROWS_PT // ZROWS)
    def _(i):
        pltpu.sync_copy(zbuf, hist.at[pl.ds(s * ROWS_PT + i * ZROWS, ZROWS)])

    plsc.subcore_barrier()

    @pl.loop(0, NGRP)
    def _(g):
        pltpu.sync_copy(edges.at[c, s * NGRP + g], sbuf)
        descs = [
            pltpu.async_copy(ones, hist.at[sbuf.at[j]], ssem, add=True)
            for j in range(GRP)
        ]
        for d in descs:
            d.wait()

    plsc.subcore_barrier()

    @pl.loop(0, ROWS_PT // ZROWS)
    def _(i):
        r0 = s * ROWS_PT + i * ZROWS
        pltpu.sync_copy(hist.at[pl.ds(r0, ZROWS)], degs.at[c, pl.ds(r0, ZROWS)])


@functools.partial(
    pl.kernel,
    out_type=jax.ShapeDtypeStruct((NC, NPAD, 32), jnp.float32),
    mesh=_mesh,
    scratch_types=[
        pltpu.VMEM_SHARED((NPAD, 32), jnp.float32),
        pltpu.VMEM((GRP, CH), jnp.int32),
        pltpu.VMEM((GRP, CH), jnp.int32),
        pltpu.VMEM((NBUF, CH, 32), jnp.float32),
        pltpu.SemaphoreType.DMA((NBUF,)),
        pltpu.SemaphoreType.DMA((SBUF,)),
    ],
    compiler_params=_sc_params,
)
def _gs_kernel(hi, edges, agg_out, agg, sbuf, dbuf, gbuf,
               gsem, ssem):
    c = lax.axis_index("c")
    s = lax.axis_index("s")
    zero = jnp.zeros((L,), jnp.float32)

    @pl.loop(0, CH)
    def _(i):
        gbuf[0, i, pl.ds(0, L)] = zero
        gbuf[0, i, pl.ds(L, L)] = zero

    @pl.loop(0, ROWS_PT // CH)
    def _(i):
        pltpu.sync_copy(gbuf.at[0],
                        agg.at[pl.ds(s * ROWS_PT + i * CH, CH)])

    plsc.subcore_barrier()

    @pl.loop(0, NGRP)
    def _(g):
        pltpu.sync_copy(edges.at[0, s * NGRP + g], sbuf)
        pltpu.sync_copy(edges.at[1, s * NGRP + g], dbuf)

        gdesc = {}
        sdesc = {}

        def start_gather(q):
            bq = q % NBUF
            gdesc[q] = pltpu.async_copy(hi.at[c].at[sbuf.at[q]], gbuf.at[bq],
                                        gsem.at[bq])

        for q in range(LEAD):
            start_gather(q)
        for j in range(GRP):
            if j - SBUF >= 0:
                sdesc[j - SBUF].wait()
            q = j + LEAD
            if q < GRP:
                start_gather(q)
            gdesc[j].wait()
            sdesc[j] = pltpu.async_copy(gbuf.at[j % NBUF],
                                        agg.at[dbuf.at[j]],
                                        ssem.at[j % SBUF], add=True)
        for j in range(GRP - SBUF, GRP):
            sdesc[j].wait()

    plsc.subcore_barrier()

    @pl.loop(0, ROWS_PT // ZROWS)
    def _(i):
        r0 = s * ROWS_PT + i * ZROWS
        pltpu.sync_copy(agg.at[pl.ds(r0, ZROWS)],
                        agg_out.at[c, pl.ds(r0, ZROWS)])


def _mm_body(x_ref, w_ref, o_ref):
    o_ref[...] = jnp.dot(x_ref[...], w_ref[...],
                         preferred_element_type=jnp.float32)


def _scale_body(y_ref, d_ref, o_ref):
    deg = d_ref[0, :, 0:1]
    norm = jnp.where(deg > 0, lax.rsqrt(jnp.maximum(deg, 1e-12)), 0.0)
    h = y_ref[...] * norm
    o_ref[0] = h[:, 0:32]
    o_ref[1] = h[:, 32:64]


def _out_body(a_ref, d_ref, b_ref, o_ref):
    agg = jnp.concatenate([a_ref[0], a_ref[1]], axis=-1)
    deg = d_ref[0, :, 0:1]
    norm = jnp.where(deg > 0, lax.rsqrt(jnp.maximum(deg, 1e-12)), 0.0)
    o_ref[...] = agg * norm + b_ref[...]


def kernel(node_embeddings, W, b, edge_index):
    pad_vals = jnp.array([[SRC_PAD], [TRASH]], jnp.int32)
    base = jnp.broadcast_to(pad_vals, (2, EPAD))
    edges2 = lax.dynamic_update_slice(base, edge_index, (0, 0)).reshape(
        2, NS * NGRP, GRP, CH)

    degs = _hist_kernel(edges2)

    y = pl.pallas_call(
        _mm_body,
        grid=(GRID,),
        in_specs=[
            pl.BlockSpec((BLK, D), lambda i: (i, 0)),
            pl.BlockSpec((D, D), lambda i: (0, 0)),
        ],
        out_specs=pl.BlockSpec((BLK, D), lambda i: (i, 0)),
        out_shape=jax.ShapeDtypeStruct((HROWS, D), jnp.float32),
    )(node_embeddings, W)

    hi = pl.pallas_call(
        _scale_body,
        grid=(GRID,),
        in_specs=[
            pl.BlockSpec((BLK, D), lambda i: (i, 0)),
            pl.BlockSpec((1, BLK, 16), lambda i: (0, i, 0)),
        ],
        out_specs=pl.BlockSpec((2, BLK, 32), lambda i: (0, i, 0)),
        out_shape=jax.ShapeDtypeStruct((2, HROWS, 32), jnp.float32),
    )(y, degs)

    aggi = _gs_kernel(hi, edges2)

    out = pl.pallas_call(
        _out_body,
        grid=(GRID,),
        in_specs=[
            pl.BlockSpec((2, BLK, 32), lambda i: (0, i, 0)),
            pl.BlockSpec((1, BLK, 16), lambda i: (1, i, 0)),
            pl.BlockSpec((1, D), lambda i: (0, 0)),
        ],
        out_specs=pl.BlockSpec((BLK, D), lambda i: (i, 0)),
        out_shape=jax.ShapeDtypeStruct((N, D), jnp.float32),
    )(aggi, degs, b.reshape(1, D))
    return out


# speedup vs baseline: 7.2151x; 1.0198x over previous
import functools

import jax
import jax.numpy as jnp
from jax import lax
from jax.experimental import pallas as pl
from jax.experimental.pallas import tpu as pltpu
from jax.experimental.pallas import tpu_sc as plsc

N = 50000
E = 800000
D = 64

NC = 2
NS = 16
L = 16

BLK = 6272
GRID = 8
HROWS = GRID * BLK
NPAD = 51200
ROWS_PT = NPAD // NS
TRASH = NPAD - 1
SRC_PAD = HROWS - 1
CH = 128
GRP = 10
NGRP = 40
EPT = CH * GRP * NGRP
EPAD = EPT * NS
ZROWS = 320
NBUF = 6
LEAD = 3
SBUF = 3

_mesh = plsc.VectorSubcoreMesh(core_axis_name="c", subcore_axis_name="s")
_sc_params = pltpu.CompilerParams(use_tc_tiling_on_sc=False,
                                 needs_layout_passes=False)


@functools.partial(
    pl.kernel,
    out_type=jax.ShapeDtypeStruct((NC, NPAD, 16), jnp.float32),
    mesh=_mesh,
    scratch_types=[
        pltpu.VMEM_SHARED((NPAD, 16), jnp.float32),
        pltpu.VMEM((GRP, CH), jnp.int32),
        pltpu.VMEM((CH, 16), jnp.float32),
        pltpu.VMEM((ZROWS, 16), jnp.float32),
        pltpu.SemaphoreType.DMA,
    ],
    compiler_params=_sc_params,
)
def _hist_kernel(edges, degs, hist, sbuf, ones, zbuf, ssem):
    c = lax.axis_index("c")
    s = lax.axis_index("s")
    one = jnp.ones((L,), jnp.float32)
    zero = jnp.zeros((L,), jnp.float32)

    @pl.loop(0, CH)
    def _(i):
        ones[i, :] = one

    @pl.loop(0, ZROWS)
    def _(i):
        zbuf[i, :] = zero

    @pl.loop(0, ROWS_PT // ZROWS)
    def _(i):
        pltpu.sync_copy(zbuf, hist.at[pl.ds(s * ROWS_PT + i * ZROWS, ZROWS)])

    plsc.subcore_barrier()

    @pl.loop(0, NGRP)
    def _(g):
        pltpu.sync_copy(edges.at[c, s * NGRP + g], sbuf)
        descs = [
            pltpu.async_copy(ones, hist.at[sbuf.at[j]], ssem, add=True)
            for j in range(GRP)
        ]
        for d in descs:
            d.wait()

    plsc.subcore_barrier()

    @pl.loop(0, ROWS_PT // ZROWS)
    def _(i):
        r0 = s * ROWS_PT + i * ZROWS
        pltpu.sync_copy(hist.at[pl.ds(r0, ZROWS)], degs.at[c, pl.ds(r0, ZROWS)])


@functools.partial(
    pl.kernel,
    out_type=jax.ShapeDtypeStruct((NC, NPAD, 32), jnp.float32),
    mesh=_mesh,
    scratch_types=[
        pltpu.VMEM_SHARED((NPAD, 32), jnp.float32),
        pltpu.VMEM((GRP, CH), jnp.int32),
        pltpu.VMEM((GRP, CH), jnp.int32),
        pltpu.VMEM((NBUF, CH, 16), jnp.int32),
        pltpu.VMEM((SBUF, CH, 32), jnp.float32),
        pltpu.SemaphoreType.DMA((NBUF,)),
        pltpu.SemaphoreType.DMA((SBUF,)),
    ],
    compiler_params=_sc_params,
)
def _gs_kernel(hi, edges, agg_out, agg, sbuf, dbuf, gbuf, fbuf,
               gsem, ssem):
    c = lax.axis_index("c")
    s = lax.axis_index("s")
    zero = jnp.zeros((L,), jnp.float32)

    @pl.loop(0, CH)
    def _(i):
        fbuf[0, i, pl.ds(0, L)] = zero
        fbuf[0, i, pl.ds(L, L)] = zero

    @pl.loop(0, ROWS_PT // CH)
    def _(i):
        pltpu.sync_copy(fbuf.at[0],
                        agg.at[pl.ds(s * ROWS_PT + i * CH, CH)])

    plsc.subcore_barrier()

    @pl.loop(0, NGRP)
    def _(g):
        pltpu.sync_copy(edges.at[0, s * NGRP + g], sbuf)
        pltpu.sync_copy(edges.at[1, s * NGRP + g], dbuf)

        gdesc = {}
        sdesc = {}

        def start_gather(q):
            bq = q % NBUF
            gdesc[q] = pltpu.async_copy(hi.at[c].at[sbuf.at[q]], gbuf.at[bq],
                                        gsem.at[bq])

        for q in range(LEAD):
            start_gather(q)
        for j in range(GRP):
            q = j + LEAD
            if q < GRP:
                start_gather(q)
            gdesc[j].wait()
            if j - SBUF >= 0:
                sdesc[j - SBUF].wait()
            bq = j % NBUF
            sb = j % SBUF

            @pl.loop(0, CH)
            def _(r, bq=bq, sb=sb):
                w = gbuf[bq, r, :]
                fbuf[sb, r, pl.ds(0, L)] = plsc.bitcast(
                    w << 16, jnp.float32)
                fbuf[sb, r, pl.ds(L, L)] = plsc.bitcast(
                    w & jnp.int32(-65536), jnp.float32)

            sdesc[j] = pltpu.async_copy(fbuf.at[sb],
                                        agg.at[dbuf.at[j]],
                                        ssem.at[sb], add=True)
        for j in range(GRP - SBUF, GRP):
            sdesc[j].wait()

    plsc.subcore_barrier()

    @pl.loop(0, ROWS_PT // ZROWS)
    def _(i):
        r0 = s * ROWS_PT + i * ZROWS
        pltpu.sync_copy(agg.at[pl.ds(r0, ZROWS)],
                        agg_out.at[c, pl.ds(r0, ZROWS)])


def _mm_body(x_ref, w_ref, o_ref):
    o_ref[...] = jnp.dot(x_ref[...], w_ref[...],
                         preferred_element_type=jnp.float32)


def _scale_body(y_ref, d_ref, o_ref):
    deg = d_ref[0, :, 0:1]
    norm = jnp.where(deg > 0, lax.rsqrt(jnp.maximum(deg, 1e-12)), 0.0)
    h = y_ref[...] * norm
    u = lax.bitcast_convert_type(h, jnp.int32)
    r = u + jnp.int32(0x7FFF) + ((u >> 16) & 1)
    bf = lax.shift_right_logical(r, 16)
    o_ref[0] = bf[:, 0:16] | (bf[:, 16:32] << 16)
    o_ref[1] = bf[:, 32:48] | (bf[:, 48:64] << 16)


def _out_body(a_ref, d_ref, b_ref, o_ref):
    agg = jnp.concatenate([a_ref[0], a_ref[1]], axis=-1)
    deg = d_ref[0, :, 0:1]
    norm = jnp.where(deg > 0, lax.rsqrt(jnp.maximum(deg, 1e-12)), 0.0)
    o_ref[...] = agg * norm + b_ref[...]


def kernel(node_embeddings, W, b, edge_index):
    pad_vals = jnp.array([[SRC_PAD], [TRASH]], jnp.int32)
    base = jnp.broadcast_to(pad_vals, (2, EPAD))
    edges2 = lax.dynamic_update_slice(base, edge_index, (0, 0)).reshape(
        2, NS * NGRP, GRP, CH)

    degs = _hist_kernel(edges2)

    y = pl.pallas_call(
        _mm_body,
        grid=(GRID,),
        in_specs=[
            pl.BlockSpec((BLK, D), lambda i: (i, 0)),
            pl.BlockSpec((D, D), lambda i: (0, 0)),
        ],
        out_specs=pl.BlockSpec((BLK, D), lambda i: (i, 0)),
        out_shape=jax.ShapeDtypeStruct((HROWS, D), jnp.float32),
    )(node_embeddings, W)

    hi = pl.pallas_call(
        _scale_body,
        grid=(GRID,),
        in_specs=[
            pl.BlockSpec((BLK, D), lambda i: (i, 0)),
            pl.BlockSpec((1, BLK, 16), lambda i: (0, i, 0)),
        ],
        out_specs=pl.BlockSpec((2, BLK, 16), lambda i: (0, i, 0)),
        out_shape=jax.ShapeDtypeStruct((2, HROWS, 16), jnp.int32),
    )(y, degs)

    aggi = _gs_kernel(hi, edges2)

    out = pl.pallas_call(
        _out_body,
        grid=(GRID,),
        in_specs=[
            pl.BlockSpec((2, BLK, 32), lambda i: (0, i, 0)),
            pl.BlockSpec((1, BLK, 16), lambda i: (1, i, 0)),
            pl.BlockSpec((1, D), lambda i: (0, 0)),
        ],
        out_specs=pl.BlockSpec((BLK, D), lambda i: (i, 0)),
        out_shape=jax.ShapeDtypeStruct((N, D), jnp.float32),
    )(aggi, degs, b.reshape(1, D))
    return out


# speedup vs baseline: 7.5670x; 1.0488x over previous
import functools

import jax
import jax.numpy as jnp
from jax import lax
from jax.experimental import pallas as pl
from jax.experimental.pallas import tpu as pltpu
from jax.experimental.pallas import tpu_sc as plsc

N = 50000
E = 800000
D = 64

NC = 2
NS = 16
L = 16

BLK = 6272
GRID = 8
HROWS = GRID * BLK
NPAD = 51200
ROWS_PT = NPAD // NS
TRASH = NPAD - 1
SRC_PAD = HROWS - 1
CH = 128
GRP = 10
NGRP = 40
EPT = CH * GRP * NGRP
EPAD = EPT * NS
ZROWS = 320
NBUF = 6
LEAD = 3
SBUF = 3

_mesh = plsc.VectorSubcoreMesh(core_axis_name="c", subcore_axis_name="s")
_sc_params = pltpu.CompilerParams(use_tc_tiling_on_sc=False,
                                 needs_layout_passes=False)


@functools.partial(
    pl.kernel,
    out_type=jax.ShapeDtypeStruct((NC, NPAD, 16), jnp.float32),
    mesh=_mesh,
    scratch_types=[
        pltpu.VMEM_SHARED((NPAD, 16), jnp.float32),
        pltpu.VMEM((GRP, CH), jnp.int32),
        pltpu.VMEM((CH, 16), jnp.float32),
        pltpu.VMEM((ZROWS, 16), jnp.float32),
        pltpu.SemaphoreType.DMA,
    ],
    compiler_params=_sc_params,
)
def _hist_kernel(edges, degs, hist, sbuf, ones, zbuf, ssem):
    c = lax.axis_index("c")
    s = lax.axis_index("s")
    one = jnp.ones((L,), jnp.float32)
    zero = jnp.zeros((L,), jnp.float32)

    @pl.loop(0, CH)
    def _(i):
        ones[i, :] = one

    @pl.loop(0, ZROWS)
    def _(i):
        zbuf[i, :] = zero

    @pl.loop(0, ROWS_PT // ZROWS)
    def _(i):
        pltpu.sync_copy(zbuf, hist.at[pl.ds(s * ROWS_PT + i * ZROWS, ZROWS)])

    plsc.subcore_barrier()

    @pl.loop(0, NGRP)
    def _(g):
        pltpu.sync_copy(edges.at[c, s * NGRP + g], sbuf)
        descs = [
            pltpu.async_copy(ones, hist.at[sbuf.at[j]], ssem, add=True)
            for j in range(GRP)
        ]
        for d in descs:
            d.wait()

    plsc.subcore_barrier()

    @pl.loop(0, ROWS_PT // ZROWS)
    def _(i):
        r0 = s * ROWS_PT + i * ZROWS
        pltpu.sync_copy(hist.at[pl.ds(r0, ZROWS)], degs.at[c, pl.ds(r0, ZROWS)])


@functools.partial(
    pl.kernel,
    out_type=jax.ShapeDtypeStruct((NC, NPAD, 128), jnp.float32),
    mesh=_mesh,
    scratch_types=[
        pltpu.VMEM_SHARED((NPAD, 32), jnp.float32),
        pltpu.VMEM((GRP, CH), jnp.int32),
        pltpu.VMEM((GRP, CH), jnp.int32),
        pltpu.VMEM((NBUF, CH, 16), jnp.int32),
        pltpu.VMEM((SBUF, CH, 32), jnp.float32),
        pltpu.SemaphoreType.DMA((NBUF,)),
        pltpu.SemaphoreType.DMA((SBUF,)),
    ],
    compiler_params=_sc_params,
)
def _gs_kernel(hi, edges, agg_out, agg, sbuf, dbuf, gbuf, fbuf,
               gsem, ssem):
    c = lax.axis_index("c")
    s = lax.axis_index("s")
    zero = jnp.zeros((L,), jnp.float32)

    @pl.loop(0, CH)
    def _(i):
        fbuf[0, i, pl.ds(0, L)] = zero
        fbuf[0, i, pl.ds(L, L)] = zero

    @pl.loop(0, ROWS_PT // CH)
    def _(i):
        pltpu.sync_copy(fbuf.at[0],
                        agg.at[pl.ds(s * ROWS_PT + i * CH, CH)])

    plsc.subcore_barrier()

    @pl.loop(0, NGRP)
    def _(g):
        pltpu.sync_copy(edges.at[0, s * NGRP + g], sbuf)
        pltpu.sync_copy(edges.at[1, s * NGRP + g], dbuf)

        gdesc = {}
        sdesc = {}

        def start_gather(q):
            bq = q % NBUF
            gdesc[q] = pltpu.async_copy(hi.at[c].at[sbuf.at[q]], gbuf.at[bq],
                                        gsem.at[bq])

        for q in range(LEAD):
            start_gather(q)
        for j in range(GRP):
            q = j + LEAD
            if q < GRP:
                start_gather(q)
            gdesc[j].wait()
            if j - SBUF >= 0:
                sdesc[j - SBUF].wait()
            bq = j % NBUF
            sb = j % SBUF

            @pl.loop(0, CH)
            def _(r, bq=bq, sb=sb):
                w = gbuf[bq, r, :]
                fbuf[sb, r, pl.ds(0, L)] = plsc.bitcast(
                    w << 16, jnp.float32)
                fbuf[sb, r, pl.ds(L, L)] = plsc.bitcast(
                    w & jnp.int32(-65536), jnp.float32)

            sdesc[j] = pltpu.async_copy(fbuf.at[sb],
                                        agg.at[dbuf.at[j]],
                                        ssem.at[sb], add=True)
        for j in range(GRP - SBUF, GRP):
            sdesc[j].wait()

    plsc.subcore_barrier()

    @pl.loop(0, ROWS_PT // ZROWS)
    def _(i):
        r0 = s * ROWS_PT + i * ZROWS
        pltpu.sync_copy(agg.at[pl.ds(r0, ZROWS)],
                        agg_out.at[c, pl.ds(r0, ZROWS), pl.ds(0, 32)])


def _mm_body(x_ref, w_ref, o_ref):
    o_ref[...] = jnp.dot(x_ref[...], w_ref[...],
                         preferred_element_type=jnp.float32)


def _scale_body(y_ref, d_ref, o_ref):
    deg = d_ref[0, :, 0:1]
    norm = jnp.where(deg > 0, lax.rsqrt(jnp.maximum(deg, 1e-12)), 0.0)
    h = y_ref[...] * norm
    u = lax.bitcast_convert_type(h, jnp.int32)
    r = u + jnp.int32(0x7FFF) + ((u >> 16) & 1)
    bf = lax.shift_right_logical(r, 16)
    o_ref[0] = bf[:, 0:16] | (bf[:, 16:32] << 16)
    o_ref[1] = bf[:, 32:48] | (bf[:, 48:64] << 16)


def _out_body(a_ref, d_ref, b_ref, o_ref):
    agg = jnp.concatenate([a_ref[0, :, 0:32], a_ref[1, :, 0:32]], axis=-1)
    deg = d_ref[0, :, 0:1]
    norm = jnp.where(deg > 0, lax.rsqrt(jnp.maximum(deg, 1e-12)), 0.0)
    o_ref[...] = agg * norm + b_ref[...]


def kernel(node_embeddings, W, b, edge_index):
    pad_vals = jnp.array([[SRC_PAD], [TRASH]], jnp.int32)
    base = jnp.broadcast_to(pad_vals, (2, EPAD))
    edges2 = lax.dynamic_update_slice(base, edge_index, (0, 0)).reshape(
        2, NS * NGRP, GRP, CH)

    degs = _hist_kernel(edges2)

    y = pl.pallas_call(
        _mm_body,
        grid=(GRID,),
        in_specs=[
            pl.BlockSpec((BLK, D), lambda i: (i, 0)),
            pl.BlockSpec((D, D), lambda i: (0, 0)),
        ],
        out_specs=pl.BlockSpec((BLK, D), lambda i: (i, 0)),
        out_shape=jax.ShapeDtypeStruct((HROWS, D), jnp.float32),
    )(node_embeddings, W)

    hi = pl.pallas_call(
        _scale_body,
        grid=(GRID,),
        in_specs=[
            pl.BlockSpec((BLK, D), lambda i: (i, 0)),
            pl.BlockSpec((1, BLK, 16), lambda i: (0, i, 0)),
        ],
        out_specs=pl.BlockSpec((2, BLK, 16), lambda i: (0, i, 0)),
        out_shape=jax.ShapeDtypeStruct((2, HROWS, 16), jnp.int32),
    )(y, degs)

    aggi = _gs_kernel(hi, edges2)

    out = pl.pallas_call(
        _out_body,
        grid=(GRID,),
        in_specs=[
            pl.BlockSpec((2, BLK, 128), lambda i: (0, i, 0)),
            pl.BlockSpec((1, BLK, 16), lambda i: (1, i, 0)),
            pl.BlockSpec((1, D), lambda i: (0, 0)),
        ],
        out_specs=pl.BlockSpec((BLK, D), lambda i: (i, 0)),
        out_shape=jax.ShapeDtypeStruct((N, D), jnp.float32),
    )(aggi, degs, b.reshape(1, D))
    return out
